# baseline (speedup 1.0000x reference)
import functools

import jax
import jax.numpy as jnp
from jax import lax
from jax.experimental import pallas as pl
from jax.experimental.pallas import tpu as pltpu
from jax.experimental.pallas import tpu_sc as plsc

VOCAB = 1000
EMB = 100
B = 16384
L = 200

NC = 2
NS = 16
NW = NC * NS
ROWS_PER_W = B // NW
CHUNK = 16
CHUNKS_PER_W = ROWS_PER_W // CHUNK


XBLK = 128
NXB = ROWS_PER_W // XBLK
GRP = XBLK // CHUNK


def _sc_counts_body(xt_hbm, counts_hbm, xa, xb, c0, c1, sxa, sxb, so0, so1):
    wid = lax.axis_index("s") * NC + lax.axis_index("c")
    lane = lax.iota(jnp.int32, 16)
    ones = jnp.full((16,), 1.0, jnp.float32)
    zeros = jnp.zeros((16,), jnp.float32)
    rbase = wid * ROWS_PER_W

    xbufs = [(xa, sxa), (xb, sxb)]
    cbufs = [(c0, so0), (c1, so1)]
    x_dummy = xt_hbm.at[:, pl.ds(0, XBLK)]
    cnt_dummy = counts_hbm.at[pl.ds(0, CHUNK), :]

    def issue_x(t, buf, sem):
        pltpu.async_copy(xt_hbm.at[:, pl.ds(rbase + t * XBLK, XBLK)], buf, sem)

    issue_x(0, xa, sxa)
    issue_x(1, xb, sxb)

    for t in range(NXB):
        xbuf, xsem = xbufs[t % 2]
        pltpu.make_async_copy(x_dummy, xbuf, xsem).wait()

        for g in range(GRP):
            kk = t * GRP + g
            cbuf, osem = cbufs[kk % 2]
            if kk >= 2:
                pltpu.make_async_copy(cnt_dummy, cbuf, osem).wait()

            def zrow(r, _, cbuf=cbuf):
                def zbody(i, _):
                    cbuf[r, pl.ds(i * 16, 16)] = zeros
                    return 0

                lax.fori_loop(0, VOCAB // 16, zbody, 0, unroll=21)
                cbuf[r, pl.ds(VOCAB - 16, 16)] = zeros
                return 0

            lax.fori_loop(0, CHUNK, zrow, 0)

            def lbody(l, _, cbuf=cbuf, xbuf=xbuf, g=g):
                vals = xbuf[l, pl.ds(g * CHUNK, CHUNK)]
                plsc.addupdate_scatter(cbuf, [lane, vals], ones)
                return 0

            lax.fori_loop(0, L, lbody, 0, unroll=20)

            row0 = rbase + t * XBLK + g * CHUNK
            pltpu.async_copy(cbuf, counts_hbm.at[pl.ds(row0, CHUNK), :], osem)

        if t + 2 < NXB:
            issue_x(t + 2, xbuf, xsem)

    pltpu.make_async_copy(cnt_dummy, c0, so0).wait()
    pltpu.make_async_copy(cnt_dummy, c1, so1).wait()


_sc_counts = pl.kernel(
    _sc_counts_body,
    out_type=jax.ShapeDtypeStruct((B, VOCAB), jnp.float32),
    mesh=plsc.VectorSubcoreMesh(core_axis_name="c", subcore_axis_name="s"),
    scratch_types=[
        pltpu.VMEM((L, XBLK), jnp.int32),
        pltpu.VMEM((L, XBLK), jnp.int32),
        pltpu.VMEM((CHUNK, VOCAB), jnp.float32),
        pltpu.VMEM((CHUNK, VOCAB), jnp.float32),
        pltpu.SemaphoreType.DMA,
        pltpu.SemaphoreType.DMA,
        pltpu.SemaphoreType.DMA,
        pltpu.SemaphoreType.DMA,
    ],
    compiler_params=pltpu.CompilerParams(needs_layout_passes=False),
)


BLK = 512


def _tc_body(counts_ref, embt_ref, lint_ref, bias_ref, outt_ref):
    cnt = counts_ref[...]
    vec = lax.dot_general(cnt, embt_ref[...], (((1,), (1,)), ((), ())),
                          preferred_element_type=jnp.float32)
    vec = vec * jnp.float32(1.0 / L)
    outt = lax.dot_general(lint_ref[...], vec, (((0,), (1,)), ((), ())),
                           preferred_element_type=jnp.float32)
    outt_ref[...] = outt + bias_ref[...]


_tc_linear = pl.pallas_call(
    _tc_body,
    grid=(B // BLK,),
    in_specs=[
        pl.BlockSpec((BLK, VOCAB), lambda i: (i, 0)),
        pl.BlockSpec((EMB, VOCAB), lambda i: (0, 0)),
        pl.BlockSpec((EMB, VOCAB), lambda i: (0, 0)),
        pl.BlockSpec((VOCAB, 1), lambda i: (0, 0)),
    ],
    out_specs=pl.BlockSpec((VOCAB, BLK), lambda i: (0, i)),
    out_shape=jax.ShapeDtypeStruct((VOCAB, B), jnp.float32),
)


@jax.jit
def kernel(x, emb_weight, lin_weight, lin_bias):
    counts = _sc_counts(x.T)
    predt = _tc_linear(counts, emb_weight.T, lin_weight.T,
                       lin_bias.reshape(VOCAB, 1))
    return predt.T


# speedup vs baseline: 1.1041x; 1.1041x over previous
import functools

import jax
import jax.numpy as jnp
from jax import lax
from jax.experimental import pallas as pl
from jax.experimental.pallas import tpu as pltpu
from jax.experimental.pallas import tpu_sc as plsc

VOCAB = 1000
EMB = 100
B = 16384
L = 200

NC = 2
NS = 16
NW = NC * NS
ROWS_PER_W = B // NW
CHUNK = 16
CHUNKS_PER_W = ROWS_PER_W // CHUNK


XBLK = 128
GRP = XBLK // CHUNK
NHALF = 2
BH = B // NHALF
RPW_H = BH // NW
NXB = RPW_H // XBLK


def _sc_counts_body(half, xt_hbm, counts_hbm, xa, xb, c0, c1, sxa, sxb, so0, so1):
    wid = lax.axis_index("s") * NC + lax.axis_index("c")
    lane = lax.iota(jnp.int32, 16)
    ones = jnp.full((16,), 1.0, jnp.float32)
    zeros = jnp.zeros((16,), jnp.float32)
    rbase = wid * RPW_H
    xbase = half * BH + rbase

    xbufs = [(xa, sxa), (xb, sxb)]
    cbufs = [(c0, so0), (c1, so1)]
    x_dummy = xt_hbm.at[:, pl.ds(0, XBLK)]
    cnt_dummy = counts_hbm.at[pl.ds(0, CHUNK), :]

    def issue_x(t, buf, sem):
        pltpu.async_copy(xt_hbm.at[:, pl.ds(xbase + t * XBLK, XBLK)], buf, sem)

    issue_x(0, xa, sxa)
    issue_x(1, xb, sxb)

    for t in range(NXB):
        xbuf, xsem = xbufs[t % 2]
        pltpu.make_async_copy(x_dummy, xbuf, xsem).wait()

        for g in range(GRP):
            kk = t * GRP + g
            cbuf, osem = cbufs[kk % 2]
            if kk >= 2:
                pltpu.make_async_copy(cnt_dummy, cbuf, osem).wait()

            def zrow(r, _, cbuf=cbuf):
                def zbody(i, _):
                    cbuf[r, pl.ds(i * 16, 16)] = zeros
                    return 0

                lax.fori_loop(0, VOCAB // 16, zbody, 0, unroll=21)
                cbuf[r, pl.ds(VOCAB - 16, 16)] = zeros
                return 0

            lax.fori_loop(0, CHUNK, zrow, 0)

            def lbody(l, _, cbuf=cbuf, xbuf=xbuf, g=g):
                vals = xbuf[l, pl.ds(g * CHUNK, CHUNK)]
                plsc.addupdate_scatter(cbuf, [lane, vals], ones)
                return 0

            lax.fori_loop(0, L, lbody, 0, unroll=20)

            row0 = rbase + t * XBLK + g * CHUNK
            pltpu.async_copy(cbuf, counts_hbm.at[pl.ds(row0, CHUNK), :], osem)

        if t + 2 < NXB:
            issue_x(t + 2, xbuf, xsem)

    pltpu.make_async_copy(cnt_dummy, c0, so0).wait()
    pltpu.make_async_copy(cnt_dummy, c1, so1).wait()


def _make_sc_half(half):
    return pl.kernel(
        functools.partial(_sc_counts_body, half),
        out_type=jax.ShapeDtypeStruct((BH, VOCAB), jnp.float32),
        mesh=plsc.VectorSubcoreMesh(core_axis_name="c", subcore_axis_name="s"),
        scratch_types=[
            pltpu.VMEM((L, XBLK), jnp.int32),
            pltpu.VMEM((L, XBLK), jnp.int32),
            pltpu.VMEM((CHUNK, VOCAB), jnp.float32),
            pltpu.VMEM((CHUNK, VOCAB), jnp.float32),
            pltpu.SemaphoreType.DMA,
            pltpu.SemaphoreType.DMA,
            pltpu.SemaphoreType.DMA,
            pltpu.SemaphoreType.DMA,
        ],
        compiler_params=pltpu.CompilerParams(needs_layout_passes=False),
    )


_sc_halves = [_make_sc_half(h) for h in range(NHALF)]


BLK = 512


def _tc_compute(counts_ref, embt_ref, lint_ref, bias_ref, outt_ref):
    cnt = counts_ref[...]
    vec = lax.dot_general(cnt, embt_ref[...], (((1,), (1,)), ((), ())),
                          preferred_element_type=jnp.float32)
    vec = vec * jnp.float32(1.0 / L)
    outt = lax.dot_general(lint_ref[...], vec, (((0,), (1,)), ((), ())),
                           preferred_element_type=jnp.float32)
    outt_ref[...] = outt + bias_ref[...]


def _tc_body0(counts_ref, embt_ref, lint_ref, bias_ref, outt_ref):
    _tc_compute(counts_ref, embt_ref, lint_ref, bias_ref, outt_ref)


def _tc_body1(counts_ref, embt_ref, lint_ref, bias_ref, prev_ref, outt_ref):
    del prev_ref
    _tc_compute(counts_ref, embt_ref, lint_ref, bias_ref, outt_ref)


_GRID_H = BH // BLK

_tc_half0 = pl.pallas_call(
    _tc_body0,
    grid=(_GRID_H,),
    in_specs=[
        pl.BlockSpec((BLK, VOCAB), lambda i: (i, 0)),
        pl.BlockSpec((EMB, VOCAB), lambda i: (0, 0)),
        pl.BlockSpec((EMB, VOCAB), lambda i: (0, 0)),
        pl.BlockSpec((VOCAB, 1), lambda i: (0, 0)),
    ],
    out_specs=pl.BlockSpec((VOCAB, BLK), lambda i: (0, i)),
    out_shape=jax.ShapeDtypeStruct((VOCAB, B), jnp.float32),
)

_tc_half1 = pl.pallas_call(
    _tc_body1,
    grid=(_GRID_H,),
    in_specs=[
        pl.BlockSpec((BLK, VOCAB), lambda i: (i, 0)),
        pl.BlockSpec((EMB, VOCAB), lambda i: (0, 0)),
        pl.BlockSpec((EMB, VOCAB), lambda i: (0, 0)),
        pl.BlockSpec((VOCAB, 1), lambda i: (0, 0)),
        pl.BlockSpec(memory_space=pl.ANY),
    ],
    out_specs=pl.BlockSpec((VOCAB, BLK), lambda i: (0, i + _GRID_H)),
    out_shape=jax.ShapeDtypeStruct((VOCAB, B), jnp.float32),
    input_output_aliases={4: 0},
)


@jax.jit
def kernel(x, emb_weight, lin_weight, lin_bias):
    xt = x.T
    embt = emb_weight.T
    lint = lin_weight.T
    bias2 = lin_bias.reshape(VOCAB, 1)
    counts0 = _sc_halves[0](xt)
    counts1 = _sc_halves[1](xt)
    p0 = _tc_half0(counts0, embt, lint, bias2)
    predt = _tc_half1(counts1, embt, lint, bias2, p0)
    return predt.T


# speedup vs baseline: 1.1071x; 1.0027x over previous
import functools

import jax
import jax.numpy as jnp
from jax import lax
from jax.experimental import pallas as pl
from jax.experimental.pallas import tpu as pltpu
from jax.experimental.pallas import tpu_sc as plsc

VOCAB = 1000
EMB = 100
B = 16384
L = 200

NC = 2
NS = 16
NW = NC * NS
ROWS_PER_W = B // NW
CHUNK = 16
CHUNKS_PER_W = ROWS_PER_W // CHUNK


XBLK = 128
GRP = XBLK // CHUNK
NHALF = 2
BH = B // NHALF
RPW_H = BH // NW
NXB = RPW_H // XBLK


def _sc_counts_body(half, xt_hbm, counts_hbm, xa, xb, c0, c1, sxa, sxb, so0, so1):
    wid = lax.axis_index("s") * NC + lax.axis_index("c")
    lane = lax.iota(jnp.int32, 16)
    ones = jnp.full((16,), 1.0, jnp.float32)
    zeros = jnp.zeros((16,), jnp.float32)
    rbase = wid * RPW_H
    xbase = half * BH + rbase

    xbufs = [(xa, sxa), (xb, sxb)]
    cbufs = [(c0, so0), (c1, so1)]
    x_dummy = xt_hbm.at[:, pl.ds(0, XBLK)]
    cnt_dummy = counts_hbm.at[pl.ds(0, CHUNK), :]

    def issue_x(t, buf, sem):
        pltpu.async_copy(xt_hbm.at[:, pl.ds(xbase + t * XBLK, XBLK)], buf, sem)

    issue_x(0, xa, sxa)
    issue_x(1, xb, sxb)

    for t in range(NXB):
        xbuf, xsem = xbufs[t % 2]
        pltpu.make_async_copy(x_dummy, xbuf, xsem).wait()

        for g in range(GRP):
            kk = t * GRP + g
            cbuf, osem = cbufs[kk % 2]
            if kk >= 2:
                pltpu.make_async_copy(cnt_dummy, cbuf, osem).wait()

            def zrow(r, _, cbuf=cbuf):
                def zbody(i, _):
                    cbuf[r, pl.ds(i * 16, 16)] = zeros
                    return 0

                lax.fori_loop(0, VOCAB // 16, zbody, 0, unroll=21)
                cbuf[r, pl.ds(VOCAB - 16, 16)] = zeros
                return 0

            lax.fori_loop(0, CHUNK, zrow, 0)

            def lbody(l, _, cbuf=cbuf, xbuf=xbuf, g=g):
                vals = xbuf[l, pl.ds(g * CHUNK, CHUNK)]
                plsc.addupdate_scatter(cbuf, [lane, vals], ones)
                return 0

            lax.fori_loop(0, L, lbody, 0, unroll=20)

            row0 = rbase + t * XBLK + g * CHUNK
            pltpu.async_copy(cbuf, counts_hbm.at[pl.ds(row0, CHUNK), :], osem)

        if t + 2 < NXB:
            issue_x(t + 2, xbuf, xsem)

    pltpu.make_async_copy(cnt_dummy, c0, so0).wait()
    pltpu.make_async_copy(cnt_dummy, c1, so1).wait()


def _make_sc_half(half):
    return pl.kernel(
        functools.partial(_sc_counts_body, half),
        out_type=jax.ShapeDtypeStruct((BH, VOCAB), jnp.float32),
        mesh=plsc.VectorSubcoreMesh(core_axis_name="c", subcore_axis_name="s"),
        scratch_types=[
            pltpu.VMEM((L, XBLK), jnp.int32),
            pltpu.VMEM((L, XBLK), jnp.int32),
            pltpu.VMEM((CHUNK, VOCAB), jnp.float32),
            pltpu.VMEM((CHUNK, VOCAB), jnp.float32),
            pltpu.SemaphoreType.DMA,
            pltpu.SemaphoreType.DMA,
            pltpu.SemaphoreType.DMA,
            pltpu.SemaphoreType.DMA,
        ],
        compiler_params=pltpu.CompilerParams(needs_layout_passes=False),
    )


_sc_halves = [_make_sc_half(h) for h in range(NHALF)]


BLK = 512


def _tc_compute(counts_ref, embt_ref, lint_ref, bias_ref, outt_ref):
    cnt = counts_ref[...].astype(jnp.bfloat16)
    vec = lax.dot_general(cnt, embt_ref[...], (((1,), (1,)), ((), ())),
                          preferred_element_type=jnp.float32)
    vec = (vec * jnp.float32(1.0 / L)).astype(jnp.bfloat16)
    outt = lax.dot_general(lint_ref[...], vec, (((0,), (1,)), ((), ())),
                           preferred_element_type=jnp.float32)
    outt_ref[...] = outt + bias_ref[...]


def _tc_body0(counts_ref, embt_ref, lint_ref, bias_ref, outt_ref):
    _tc_compute(counts_ref, embt_ref, lint_ref, bias_ref, outt_ref)


def _tc_body1(counts_ref, embt_ref, lint_ref, bias_ref, prev_ref, outt_ref):
    del prev_ref
    _tc_compute(counts_ref, embt_ref, lint_ref, bias_ref, outt_ref)


_GRID_H = BH // BLK

_tc_half0 = pl.pallas_call(
    _tc_body0,
    grid=(_GRID_H,),
    in_specs=[
        pl.BlockSpec((BLK, VOCAB), lambda i: (i, 0)),
        pl.BlockSpec((EMB, VOCAB), lambda i: (0, 0)),
        pl.BlockSpec((EMB, VOCAB), lambda i: (0, 0)),
        pl.BlockSpec((VOCAB, 1), lambda i: (0, 0)),
    ],
    out_specs=pl.BlockSpec((VOCAB, BLK), lambda i: (0, i)),
    out_shape=jax.ShapeDtypeStruct((VOCAB, B), jnp.float32),
)

_tc_half1 = pl.pallas_call(
    _tc_body1,
    grid=(_GRID_H,),
    in_specs=[
        pl.BlockSpec((BLK, VOCAB), lambda i: (i, 0)),
        pl.BlockSpec((EMB, VOCAB), lambda i: (0, 0)),
        pl.BlockSpec((EMB, VOCAB), lambda i: (0, 0)),
        pl.BlockSpec((VOCAB, 1), lambda i: (0, 0)),
        pl.BlockSpec(memory_space=pl.ANY),
    ],
    out_specs=pl.BlockSpec((VOCAB, BLK), lambda i: (0, i + _GRID_H)),
    out_shape=jax.ShapeDtypeStruct((VOCAB, B), jnp.float32),
    input_output_aliases={4: 0},
)


@jax.jit
def kernel(x, emb_weight, lin_weight, lin_bias):
    xt = x.T
    embt = emb_weight.T.astype(jnp.bfloat16)
    lint = lin_weight.T.astype(jnp.bfloat16)
    bias2 = lin_bias.reshape(VOCAB, 1)
    counts0 = _sc_halves[0](xt)
    counts1 = _sc_halves[1](xt)
    p0 = _tc_half0(counts0, embt, lint, bias2)
    predt = _tc_half1(counts1, embt, lint, bias2, p0)
    return predt.T
